# baseline (speedup 1.0000x reference)
import functools

import jax
import jax.numpy as jnp
from jax import lax
from jax.experimental import pallas as pl
from jax.experimental.pallas import tpu as pltpu
from jax.experimental.pallas import tpu_sc as plsc

NC = 2
NS = 16
L = 16


def _lse_body(t_ref, o_ref):
    t = t_ref[...]
    m = jnp.max(t, axis=1, keepdims=True)
    s = jnp.sum(jnp.exp(t - m), axis=1, keepdims=True)
    o_ref[...] = (m + jnp.log(s))[:, 0]


def _row_lse(table):
    v, c = table.shape
    return pl.pallas_call(
        _lse_body,
        out_shape=jax.ShapeDtypeStruct((v,), jnp.float32),
    )(table)


def _make_mesh():
    return plsc.VectorSubcoreMesh(
        core_axis_name="c", subcore_axis_name="s", num_cores=NC, num_subcores=NS
    )


def _make_sc_main(b, t, t_pad, vocab, dim, dim_pad):
    n_workers = NC * NS
    b_per_w = b // n_workers
    per_w = b_per_w * t
    per_wp = b_per_w * t_pad

    @functools.partial(
        pl.kernel,
        out_type=[
            jax.ShapeDtypeStruct((b, t_pad, dim_pad), jnp.float32),
            jax.ShapeDtypeStruct((n_workers, L), jnp.float32),
        ],
        mesh=_make_mesh(),
        scratch_types=[
            pltpu.VMEM((per_wp,), jnp.int32),
            pltpu.VMEM((per_w,), jnp.int32),
            pltpu.VMEM((per_w,), jnp.int32),
            pltpu.VMEM((per_w,), jnp.int32),
            pltpu.VMEM((per_w,), jnp.float32),
            pltpu.VMEM((per_w,), jnp.float32),
            pltpu.VMEM((L,), jnp.float32),
            pltpu.VMEM((t_pad, dim_pad), jnp.float32),
            pltpu.VMEM((t_pad, dim_pad), jnp.float32),
            pltpu.SemaphoreType.DMA,
            pltpu.SemaphoreType.DMA,
            pltpu.SemaphoreType.DMA,
            pltpu.SemaphoreType.DMA,
            pltpu.SemaphoreType.DMA,
            pltpu.SemaphoreType.DMA,
        ],
    )
    def sc_main(xpf_hbm, xf_hbm, tf_hbm, tabpad_hbm, tabflat_hbm, lse_hbm,
                out_hbm, part_hbm, idxp_v, idx_v, tgt_v, pick_v, pval_v,
                lval_v, acc_v, rows0_v, rows1_v, gsem0, gsem1, osem0, osem1,
                psem, lsem):
        wid = lax.axis_index("s") * NC + lax.axis_index("c")
        bbase = wid * b_per_w
        bufs = (rows0_v, rows1_v)
        gsems = (gsem0, gsem1)
        osems = (osem0, osem1)

        pltpu.sync_copy(xpf_hbm.at[pl.ds(wid * per_wp, per_wp)], idxp_v)
        pltpu.sync_copy(xf_hbm.at[pl.ds(wid * per_w, per_w)], idx_v)
        pltpu.sync_copy(tf_hbm.at[pl.ds(wid * per_w, per_w)], tgt_v)

        def pick_body(i, _):
            o = i * L
            pick_v[pl.ds(o, L)] = idx_v[pl.ds(o, L)] * dim + tgt_v[pl.ds(o, L)]
            return 0

        lax.fori_loop(0, per_w // L, pick_body, 0)
        pick_dma = pltpu.async_copy(tabflat_hbm.at[pick_v], pval_v, psem)
        lse_dma = pltpu.async_copy(lse_hbm.at[idx_v], lval_v, lsem)

        def start_gather(k):
            p = k % 2
            return pltpu.async_copy(
                tabpad_hbm.at[idxp_v.at[pl.ds(k * t_pad, t_pad)]],
                bufs[p], gsems[p])

        gd = {0: start_gather(0)}
        od = {}

        for k in range(b_per_w):
            p = k % 2
            gd.pop(k).wait()
            od[k] = pltpu.async_copy(bufs[p], out_hbm.at[bbase + k], osems[p])
            if k + 1 < b_per_w:
                if k - 1 in od:
                    od.pop(k - 1).wait()
                gd[k + 1] = start_gather(k + 1)
        od.pop(b_per_w - 1).wait()

        pick_dma.wait()
        lse_dma.wait()

        def acc_body(i, acc):
            o = i * L
            return acc + lval_v[pl.ds(o, L)] - pval_v[pl.ds(o, L)]

        acc = lax.fori_loop(0, per_w // L, acc_body,
                            jnp.zeros((L,), jnp.float32))
        acc_v[...] = acc
        pltpu.sync_copy(acc_v, part_hbm.at[wid])

    return sc_main


def _fin_body(p_ref, o_ref, n):
    o_ref[...] = (jnp.sum(p_ref[...]) / n).reshape(1, 1)


def kernel(x, targets, next_token_table):
    b, t = x.shape
    vocab, dim = next_token_table.shape
    n_rows = b * t
    t_pad = -(-t // 8) * 8
    dim_pad = -(-dim // 128) * 128

    xf = x.reshape(-1).astype(jnp.int32)
    tf = targets.reshape(-1).astype(jnp.int32)
    tabflat = jnp.pad(next_token_table.reshape(-1), (0, 8))
    tab_pad = jnp.pad(next_token_table, ((0, 0), (0, dim_pad - dim)))
    xp = jnp.concatenate(
        [x.astype(jnp.int32), x[:, 2 * t - t_pad:].astype(jnp.int32)], axis=1)
    xpf = xp.reshape(-1)

    lse = _row_lse(next_token_table)
    out_pad, part = _make_sc_main(b, t, t_pad, vocab, dim, dim_pad)(
        xpf, xf, tf, tab_pad, tabflat, lse)
    logits = out_pad[:, :t, :dim]

    loss = pl.pallas_call(
        functools.partial(_fin_body, n=float(n_rows)),
        out_shape=jax.ShapeDtypeStruct((1, 1), jnp.float32),
    )(part)

    return logits, loss[0, 0]


# speedup vs baseline: 1.0010x; 1.0010x over previous
import functools

import jax
import jax.numpy as jnp
from jax import lax
from jax.experimental import pallas as pl
from jax.experimental.pallas import tpu as pltpu
from jax.experimental.pallas import tpu_sc as plsc

NC = 2
NS = 16
L = 16


def _lse_body(t_ref, o_ref):
    t = t_ref[...]
    m = jnp.max(t, axis=1, keepdims=True)
    s = jnp.sum(jnp.exp(t - m), axis=1, keepdims=True)
    o_ref[...] = (m + jnp.log(s))[:, 0]


def _row_lse(table):
    v, c = table.shape
    return pl.pallas_call(
        _lse_body,
        out_shape=jax.ShapeDtypeStruct((v,), jnp.float32),
    )(table)


def _make_mesh():
    return plsc.VectorSubcoreMesh(
        core_axis_name="c", subcore_axis_name="s", num_cores=NC, num_subcores=NS
    )


def _make_sc_main(b, t, t_pad, vocab, dim, dim_pad):
    n_workers = NC * NS
    b_per_w = b // n_workers
    per_w = b_per_w * t
    per_wp = b_per_w * t_pad

    @functools.partial(
        pl.kernel,
        out_type=[
            jax.ShapeDtypeStruct((b, t_pad, dim_pad), jnp.float32),
            jax.ShapeDtypeStruct((n_workers, L), jnp.float32),
        ],
        mesh=_make_mesh(),
        scratch_types=[
            pltpu.VMEM((per_wp,), jnp.int32),
            pltpu.VMEM((per_w,), jnp.int32),
            pltpu.VMEM((per_w,), jnp.int32),
            pltpu.VMEM((per_w,), jnp.int32),
            pltpu.VMEM((per_w,), jnp.float32),
            pltpu.VMEM((per_w,), jnp.float32),
            pltpu.VMEM((L,), jnp.float32),
            pltpu.VMEM((t_pad, dim_pad), jnp.float32),
            pltpu.VMEM((t_pad, dim_pad), jnp.float32),
            pltpu.SemaphoreType.DMA,
            pltpu.SemaphoreType.DMA,
            pltpu.SemaphoreType.DMA,
            pltpu.SemaphoreType.DMA,
            pltpu.SemaphoreType.DMA,
            pltpu.SemaphoreType.DMA,
        ],
    )
    def sc_main(xpf_hbm, xf_hbm, tf_hbm, tabpad_hbm, tabflat_hbm, lse_hbm,
                out_hbm, part_hbm, idxp_v, idx_v, tgt_v, pick_v, pval_v,
                lval_v, acc_v, rows0_v, rows1_v, gsem0, gsem1, osem0, osem1,
                psem, lsem):
        wid = lax.axis_index("s") * NC + lax.axis_index("c")
        bbase = wid * b_per_w
        bufs = (rows0_v, rows1_v)
        gsems = (gsem0, gsem1)
        osems = (osem0, osem1)

        pltpu.sync_copy(xpf_hbm.at[pl.ds(wid * per_wp, per_wp)], idxp_v)
        pltpu.sync_copy(xf_hbm.at[pl.ds(wid * per_w, per_w)], idx_v)
        pltpu.sync_copy(tf_hbm.at[pl.ds(wid * per_w, per_w)], tgt_v)

        def pick_body(i, _):
            o = i * L
            pick_v[pl.ds(o, L)] = idx_v[pl.ds(o, L)] * dim + tgt_v[pl.ds(o, L)]
            return 0

        lax.fori_loop(0, per_w // L, pick_body, 0)
        pick_dma = pltpu.async_copy(tabflat_hbm.at[pick_v], pval_v, psem)
        lse_dma = pltpu.async_copy(lse_hbm.at[idx_v], lval_v, lsem)

        def start_gather(k):
            p = k % 2
            return pltpu.async_copy(
                tabpad_hbm.at[idxp_v.at[pl.ds(k * t_pad, t_pad)]],
                bufs[p], gsems[p])

        gd = {0: start_gather(0)}
        od = {}

        for k in range(b_per_w):
            p = k % 2
            gd.pop(k).wait()
            od[k] = pltpu.async_copy(bufs[p], out_hbm.at[bbase + k], osems[p])
            if k + 1 < b_per_w:
                if k - 1 in od:
                    od.pop(k - 1).wait()
                gd[k + 1] = start_gather(k + 1)
        for k in sorted(od):
            od.pop(k).wait()

        pick_dma.wait()
        lse_dma.wait()

        def acc_body(i, acc):
            o = i * L
            return acc + lval_v[pl.ds(o, L)] - pval_v[pl.ds(o, L)]

        acc = lax.fori_loop(0, per_w // L, acc_body,
                            jnp.zeros((L,), jnp.float32))
        acc_v[...] = acc
        pltpu.sync_copy(acc_v, part_hbm.at[wid])

    return sc_main


def _fin_body(p_ref, o_ref, n):
    o_ref[...] = (jnp.sum(p_ref[...]) / n).reshape(1, 1)


def kernel(x, targets, next_token_table):
    b, t = x.shape
    vocab, dim = next_token_table.shape
    n_rows = b * t
    t_pad = -(-t // 8) * 8
    dim_pad = -(-dim // 128) * 128

    xf = x.reshape(-1).astype(jnp.int32)
    tf = targets.reshape(-1).astype(jnp.int32)
    tabflat = jnp.pad(next_token_table.reshape(-1), (0, 8))
    tab_pad = jnp.pad(next_token_table, ((0, 0), (0, dim_pad - dim)))
    xp = jnp.concatenate(
        [x.astype(jnp.int32), x[:, 2 * t - t_pad:].astype(jnp.int32)], axis=1)
    xpf = xp.reshape(-1)

    lse = _row_lse(next_token_table)
    out_pad, part = _make_sc_main(b, t, t_pad, vocab, dim, dim_pad)(
        xpf, xf, tf, tab_pad, tabflat, lse)
    logits = out_pad[:, :t, :dim]

    loss = pl.pallas_call(
        functools.partial(_fin_body, n=float(n_rows)),
        out_shape=jax.ShapeDtypeStruct((1, 1), jnp.float32),
    )(part)

    return logits, loss[0, 0]


# speedup vs baseline: 1.0078x; 1.0068x over previous
import functools

import jax
import jax.numpy as jnp
from jax import lax
from jax.experimental import pallas as pl
from jax.experimental.pallas import tpu as pltpu
from jax.experimental.pallas import tpu_sc as plsc

NC = 2
NS = 16
L = 16


def _lse_body(t_ref, o_ref):
    t = t_ref[...]
    m = jnp.max(t, axis=1, keepdims=True)
    s = jnp.sum(jnp.exp(t - m), axis=1, keepdims=True)
    o_ref[...] = (m + jnp.log(s))[:, 0]


def _row_lse(table):
    v, c = table.shape
    return pl.pallas_call(
        _lse_body,
        out_shape=jax.ShapeDtypeStruct((v,), jnp.float32),
    )(table)


def _make_mesh():
    return plsc.VectorSubcoreMesh(
        core_axis_name="c", subcore_axis_name="s", num_cores=NC, num_subcores=NS
    )


def _make_sc_main(b, t, t_pad, vocab, dim, dim_pad):
    n_workers = NC * NS
    b_per_w = b // n_workers
    per_w = b_per_w * t
    per_wp = b_per_w * t_pad

    @functools.partial(
        pl.kernel,
        out_type=[
            jax.ShapeDtypeStruct((b, t_pad, dim_pad), jnp.float32),
            jax.ShapeDtypeStruct((n_workers, L), jnp.float32),
        ],
        mesh=_make_mesh(),
        scratch_types=[
            pltpu.VMEM((per_wp,), jnp.int32),
            pltpu.VMEM((per_w,), jnp.int32),
            pltpu.VMEM((per_w,), jnp.int32),
            pltpu.VMEM((per_w,), jnp.int32),
            pltpu.VMEM((per_w,), jnp.float32),
            pltpu.VMEM((per_w,), jnp.float32),
            pltpu.VMEM((L,), jnp.float32),
            pltpu.VMEM((t_pad, dim_pad), jnp.float32),
            pltpu.VMEM((t_pad, dim_pad), jnp.float32),
            pltpu.SemaphoreType.DMA,
            pltpu.SemaphoreType.DMA,
            pltpu.SemaphoreType.DMA,
            pltpu.SemaphoreType.DMA,
            pltpu.SemaphoreType.DMA,
            pltpu.SemaphoreType.DMA,
        ],
    )
    def sc_main(xpf_hbm, xf_hbm, tf_hbm, ta_hbm, tb_hbm, tabflat_hbm, lse_hbm,
                out_hbm, part_hbm, idxp_v, idx_v, tgt_v, pick_v, pval_v,
                lval_v, acc_v, rows0_v, rows1_v, gsem0, gsem1, osem0, osem1,
                psem, lsem):
        wid = lax.axis_index("s") * NC + lax.axis_index("c")
        bbase = wid * b_per_w
        bufs = (rows0_v, rows1_v)
        gsems = (gsem0, gsem1)
        osems = (osem0, osem1)

        pltpu.sync_copy(xpf_hbm.at[pl.ds(wid * per_wp, per_wp)], idxp_v)
        pltpu.sync_copy(xf_hbm.at[pl.ds(wid * per_w, per_w)], idx_v)
        pltpu.sync_copy(tf_hbm.at[pl.ds(wid * per_w, per_w)], tgt_v)

        def pick_body(i, _):
            o = i * L
            pick_v[pl.ds(o, L)] = idx_v[pl.ds(o, L)] * dim + tgt_v[pl.ds(o, L)]
            return 0

        lax.fori_loop(0, per_w // L, pick_body, 0)
        pick_dma = pltpu.async_copy(tabflat_hbm.at[pick_v], pval_v, psem)
        lse_dma = pltpu.async_copy(lse_hbm.at[idx_v], lval_v, lsem)

        half = dim_pad // 2

        def start_gather(k):
            p = k % 2
            idx = idxp_v.at[pl.ds(k * t_pad, t_pad)]
            da = pltpu.async_copy(
                ta_hbm.at[idx], bufs[p].at[:, pl.ds(0, half)], gsems[p])
            db = pltpu.async_copy(
                tb_hbm.at[idx], bufs[p].at[:, pl.ds(half, half)], gsems[p])
            return (da, db)

        gd = {0: start_gather(0)}
        od = {}

        for k in range(b_per_w):
            p = k % 2
            da, db = gd.pop(k)
            da.wait()
            db.wait()
            od[k] = pltpu.async_copy(bufs[p], out_hbm.at[bbase + k], osems[p])
            if k + 1 < b_per_w:
                if k - 1 in od:
                    od.pop(k - 1).wait()
                gd[k + 1] = start_gather(k + 1)
        for k in sorted(od):
            od.pop(k).wait()

        pick_dma.wait()
        lse_dma.wait()

        def acc_body(i, acc):
            o = i * L
            return acc + lval_v[pl.ds(o, L)] - pval_v[pl.ds(o, L)]

        acc = lax.fori_loop(0, per_w // L, acc_body,
                            jnp.zeros((L,), jnp.float32))
        acc_v[...] = acc
        pltpu.sync_copy(acc_v, part_hbm.at[wid])

    return sc_main


def _fin_body(p_ref, o_ref, n):
    o_ref[...] = (jnp.sum(p_ref[...]) / n).reshape(1, 1)


def kernel(x, targets, next_token_table):
    b, t = x.shape
    vocab, dim = next_token_table.shape
    n_rows = b * t
    t_pad = -(-t // 8) * 8
    dim_pad = -(-dim // 128) * 128

    xf = x.reshape(-1).astype(jnp.int32)
    tf = targets.reshape(-1).astype(jnp.int32)
    tabflat = jnp.pad(next_token_table.reshape(-1), (0, 8))
    tab_pad = jnp.pad(next_token_table, ((0, 0), (0, dim_pad - dim)))
    tab_a = tab_pad[:, :dim_pad // 2]
    tab_b = tab_pad[:, dim_pad // 2:]
    xp = jnp.concatenate(
        [x.astype(jnp.int32), x[:, 2 * t - t_pad:].astype(jnp.int32)], axis=1)
    xpf = xp.reshape(-1)

    lse = _row_lse(next_token_table)
    out_pad, part = _make_sc_main(b, t, t_pad, vocab, dim, dim_pad)(
        xpf, xf, tf, tab_a, tab_b, tabflat, lse)
    logits = out_pad[:, :t, :dim]

    loss = pl.pallas_call(
        functools.partial(_fin_body, n=float(n_rows)),
        out_shape=jax.ShapeDtypeStruct((1, 1), jnp.float32),
    )(part)

    return logits, loss[0, 0]


# speedup vs baseline: 1.0147x; 1.0068x over previous
import functools

import jax
import jax.numpy as jnp
from jax import lax
from jax.experimental import pallas as pl
from jax.experimental.pallas import tpu as pltpu
from jax.experimental.pallas import tpu_sc as plsc

NC = 2
NS = 16
L = 16


def _lse_body(t_ref, o_ref):
    t = t_ref[...]
    m = jnp.max(t, axis=1, keepdims=True)
    s = jnp.sum(jnp.exp(t - m), axis=1, keepdims=True)
    o_ref[...] = (m + jnp.log(s))[:, 0]


def _row_lse(table):
    v, c = table.shape
    return pl.pallas_call(
        _lse_body,
        out_shape=jax.ShapeDtypeStruct((v,), jnp.float32),
    )(table)


def _make_mesh():
    return plsc.VectorSubcoreMesh(
        core_axis_name="c", subcore_axis_name="s", num_cores=NC, num_subcores=NS
    )


def _make_sc_main(b, t, t_pad, vocab, dim, dim_pad):
    n_workers = NC * NS
    b_per_w = b // n_workers
    per_w = b_per_w * t
    per_wp = b_per_w * t_pad

    @functools.partial(
        pl.kernel,
        out_type=[
            jax.ShapeDtypeStruct((b, t_pad, dim_pad), jnp.float32),
            jax.ShapeDtypeStruct((n_workers, L), jnp.float32),
        ],
        mesh=_make_mesh(),
        scratch_types=[
            pltpu.VMEM((per_wp,), jnp.int32),
            pltpu.VMEM((per_w,), jnp.int32),
            pltpu.VMEM((per_w,), jnp.int32),
            pltpu.VMEM((per_w,), jnp.int32),
            pltpu.VMEM((per_w,), jnp.float32),
            pltpu.VMEM((per_w,), jnp.float32),
            pltpu.VMEM((L,), jnp.float32),
            pltpu.VMEM((t_pad, dim_pad // 2), jnp.float32),
            pltpu.VMEM((t_pad, dim_pad // 2), jnp.float32),
            pltpu.VMEM((t_pad, dim_pad // 2), jnp.float32),
            pltpu.VMEM((t_pad, dim_pad // 2), jnp.float32),
            pltpu.SemaphoreType.DMA,
            pltpu.SemaphoreType.DMA,
            pltpu.SemaphoreType.DMA,
            pltpu.SemaphoreType.DMA,
            pltpu.SemaphoreType.DMA,
            pltpu.SemaphoreType.DMA,
            pltpu.SemaphoreType.DMA,
            pltpu.SemaphoreType.DMA,
            pltpu.SemaphoreType.DMA,
            pltpu.SemaphoreType.DMA,
        ],
    )
    def sc_main(xpf_hbm, xf_hbm, tf_hbm, ta_hbm, tb_hbm, tabflat_hbm, lse_hbm,
                out_hbm, part_hbm, idxp_v, idx_v, tgt_v, pick_v, pval_v,
                lval_v, acc_v, rows0_v, rows1_v, rows2_v, rows3_v,
                gsem0, gsem1, gsem2, gsem3, osem0, osem1, osem2, osem3,
                psem, lsem):
        wid = lax.axis_index("s") * NC + lax.axis_index("c")
        bbase = wid * b_per_w
        bufs = (rows0_v, rows1_v, rows2_v, rows3_v)
        gsems = (gsem0, gsem1, gsem2, gsem3)
        osems = (osem0, osem1, osem2, osem3)

        pltpu.sync_copy(xpf_hbm.at[pl.ds(wid * per_wp, per_wp)], idxp_v)
        pltpu.sync_copy(xf_hbm.at[pl.ds(wid * per_w, per_w)], idx_v)
        pltpu.sync_copy(tf_hbm.at[pl.ds(wid * per_w, per_w)], tgt_v)

        def pick_body(i, _):
            o = i * L
            pick_v[pl.ds(o, L)] = idx_v[pl.ds(o, L)] * dim + tgt_v[pl.ds(o, L)]
            return 0

        lax.fori_loop(0, per_w // L, pick_body, 0)
        pick_dma = pltpu.async_copy(tabflat_hbm.at[pick_v], pval_v, psem)
        lse_dma = pltpu.async_copy(lse_hbm.at[idx_v], lval_v, lsem)

        half = dim_pad // 2
        tabs = (ta_hbm, tb_hbm)
        n_chunks = 2 * b_per_w

        def start_gather(j):
            k, h = j // 2, j % 2
            idx = idxp_v.at[pl.ds(k * t_pad, t_pad)]
            return pltpu.async_copy(tabs[h].at[idx], bufs[j % 4],
                                    gsems[j % 4])

        gd = {0: start_gather(0), 1: start_gather(1)}
        od = {}

        for j in range(n_chunks):
            k, h = j // 2, j % 2
            gd.pop(j).wait()
            od[j] = pltpu.async_copy(
                bufs[j % 4], out_hbm.at[bbase + k, :, pl.ds(h * half, half)],
                osems[j % 4])
            if j + 2 < n_chunks:
                if j - 2 in od:
                    od.pop(j - 2).wait()
                gd[j + 2] = start_gather(j + 2)
        for j in sorted(od):
            od.pop(j).wait()

        pick_dma.wait()
        lse_dma.wait()

        def acc_body(i, acc):
            o = i * L
            return acc + lval_v[pl.ds(o, L)] - pval_v[pl.ds(o, L)]

        acc = lax.fori_loop(0, per_w // L, acc_body,
                            jnp.zeros((L,), jnp.float32))
        acc_v[...] = acc
        pltpu.sync_copy(acc_v, part_hbm.at[wid])

    return sc_main


def _fin_body(p_ref, o_ref, n):
    o_ref[...] = (jnp.sum(p_ref[...]) / n).reshape(1, 1)


def kernel(x, targets, next_token_table):
    b, t = x.shape
    vocab, dim = next_token_table.shape
    n_rows = b * t
    t_pad = -(-t // 8) * 8
    dim_pad = -(-dim // 128) * 128

    xf = x.reshape(-1).astype(jnp.int32)
    tf = targets.reshape(-1).astype(jnp.int32)
    tabflat = jnp.pad(next_token_table.reshape(-1), (0, 8))
    tab_pad = jnp.pad(next_token_table, ((0, 0), (0, dim_pad - dim)))
    tab_a = tab_pad[:, :dim_pad // 2]
    tab_b = tab_pad[:, dim_pad // 2:]
    xp = jnp.concatenate(
        [x.astype(jnp.int32), x[:, 2 * t - t_pad:].astype(jnp.int32)], axis=1)
    xpf = xp.reshape(-1)

    lse = _row_lse(next_token_table)
    out_pad, part = _make_sc_main(b, t, t_pad, vocab, dim, dim_pad)(
        xpf, xf, tf, tab_a, tab_b, tabflat, lse)
    logits = out_pad[:, :t, :dim]

    loss = pl.pallas_call(
        functools.partial(_fin_body, n=float(n_rows)),
        out_shape=jax.ShapeDtypeStruct((1, 1), jnp.float32),
    )(part)

    return logits, loss[0, 0]
